# baseline (speedup 1.0000x reference)
import functools

import jax
import jax.numpy as jnp
from jax import lax
from jax.experimental import pallas as pl
from jax.experimental.pallas import tpu as pltpu
from jax.experimental.pallas import tpu_sc as plsc

B = 16
L = 4096
VOCAB = 32000
DIM = 1024

NUM_CORES = 2
NUM_SUBCORES = 16
LANES = 16
NUM_WORKERS = NUM_CORES * NUM_SUBCORES
TOK_PER_WORKER = L
STEPS = TOK_PER_WORKER // LANES

VROWS = VOCAB // 128
VBLK = 50


def _rowsum_body(x_ref, o_ref):
    o_ref[0] = jnp.sum(x_ref[:], axis=2)


def _row_sums(embed_table):
    return pl.pallas_call(
        _rowsum_body,
        grid=(VROWS // VBLK,),
        in_specs=[pl.BlockSpec((VBLK, 128, DIM), lambda i: (i, 0, 0))],
        out_specs=pl.BlockSpec((1, VBLK, 128), lambda i: (i, 0, 0)),
        out_shape=jax.ShapeDtypeStruct((VROWS // VBLK, VBLK, 128), jnp.float32),
    )(embed_table.reshape(VROWS, 128, DIM))


_SC_MESH = plsc.VectorSubcoreMesh(
    core_axis_name="c", subcore_axis_name="s",
    num_cores=1, num_subcores=NUM_SUBCORES,
)


@functools.partial(
    pl.kernel,
    out_type=jax.ShapeDtypeStruct((B, LANES), jnp.float32),
    mesh=_SC_MESH,
    compiler_params=pltpu.CompilerParams(needs_layout_passes=False),
    scratch_types=[
        pltpu.VMEM((VOCAB,), jnp.float32),
        pltpu.VMEM((TOK_PER_WORKER,), jnp.int32),
        pltpu.VMEM((B,), jnp.int32),
        pltpu.VMEM((LANES,), jnp.float32),
    ],
)
def _sc_gather_sum(rs_hbm, tok_hbm, sl_hbm, out_hbm, rs_v, tok_v, sl_v, acc_v):
    wid = lax.axis_index("s")

    @pl.when(wid < B)
    def _():
        pltpu.sync_copy(rs_hbm, rs_v)
        pltpu.sync_copy(tok_hbm.at[pl.ds(wid * TOK_PER_WORKER, TOK_PER_WORKER)], tok_v)
        pltpu.sync_copy(sl_hbm, sl_v)

        limit = plsc.load_gather(sl_v, [jnp.full((LANES,), wid, jnp.int32)])
        lane_ids = lax.iota(jnp.int32, LANES)

        def body(i, acc):
            idx = tok_v[pl.ds(i * LANES, LANES)]
            pos = i * LANES + lane_ids
            idx = jnp.where(pos < limit, idx, 0)
            return acc + plsc.load_gather(rs_v, [idx])

        acc_v[...] = lax.fori_loop(
            0, STEPS, body, jnp.zeros((LANES,), jnp.float32), unroll=8
        )
        pltpu.sync_copy(acc_v, out_hbm.at[wid])


def kernel(tokens, seq_lengths, embed_table):
    row_sums = _row_sums(embed_table).reshape(VOCAB)
    partials = _sc_gather_sum(row_sums, tokens.reshape(-1), seq_lengths)
    return jnp.sum(partials)
